# Initial kernel scaffold; baseline (speedup 1.0000x reference)
import jax
import jax.numpy as jnp
from jax.experimental import pallas as pl


def kernel(x, table):
    raise NotImplementedError("write your pallas kernel here")


# speedup vs baseline: 4.8057x; 4.8057x over previous
import functools

import jax
import jax.numpy as jnp
from jax import lax
from jax.experimental import pallas as pl
from jax.experimental.pallas import tpu as pltpu
from jax.experimental.pallas import tpu_sc as plsc

EMBED = 32
IDXW = 128
ROWS_PER_CHUNK = 8
CHUNK = ROWS_PER_CHUNK * IDXW


def _emb_call(n_idx, vocab):
    n_workers = 32
    per_w = n_idx // n_workers
    n_chunks = per_w // CHUNK
    chunk_rows = ROWS_PER_CHUNK

    mesh = plsc.VectorSubcoreMesh(core_axis_name="c", subcore_axis_name="s")

    @functools.partial(
        pl.kernel,
        mesh=mesh,
        out_type=jax.ShapeDtypeStruct((n_idx, EMBED), jnp.float32),
        scratch_types=[
            pltpu.VMEM((chunk_rows, IDXW), jnp.int32),
            pltpu.VMEM((CHUNK, EMBED), jnp.float32),
            pltpu.SemaphoreType.DMA,
        ],
        compiler_params=pltpu.CompilerParams(use_tc_tiling_on_sc=False),
    )
    def emb(idx_hbm, table_hbm, out_hbm, idx_v, rows_v, sem):
        wid = lax.axis_index("s") * 2 + lax.axis_index("c")
        row_base = wid * (per_w // IDXW)

        def body(g, carry):
            row_off = row_base + g * chunk_rows
            pltpu.sync_copy(idx_hbm.at[pl.ds(row_off, chunk_rows)], idx_v)
            copies = []
            for j in range(chunk_rows):
                copies.append(
                    pltpu.async_copy(
                        table_hbm.at[idx_v.at[j]],
                        rows_v.at[pl.ds(j * IDXW, IDXW)],
                        sem,
                    )
                )
            for c in copies:
                c.wait()
            pltpu.sync_copy(
                rows_v, out_hbm.at[pl.ds(row_off * IDXW, CHUNK)]
            )
            return carry

        lax.fori_loop(0, n_chunks, body, 0)

    return emb


def kernel(x, table):
    n_idx = x.shape[0] * x.shape[1]
    idx2d = x.reshape(n_idx // IDXW, IDXW)
    out = _emb_call(n_idx, table.shape[0])(idx2d, table)
    return out.reshape(x.shape[0], x.shape[1], EMBED)


# speedup vs baseline: 5.0508x; 1.0510x over previous
import functools

import jax
import jax.numpy as jnp
from jax import lax
from jax.experimental import pallas as pl
from jax.experimental.pallas import tpu as pltpu
from jax.experimental.pallas import tpu_sc as plsc

EMBED = 32
IDXW = 128
ROWS_PER_CHUNK = 8
CHUNK = ROWS_PER_CHUNK * IDXW
N_WORKERS = 32


def _emb_call(n_idx):
    per_w = n_idx // N_WORKERS
    n_chunks = per_w // CHUNK
    rows_per_w = per_w // IDXW
    assert n_chunks % 4 == 0 and n_chunks >= 8

    mesh = plsc.VectorSubcoreMesh(core_axis_name="c", subcore_axis_name="s")

    @functools.partial(
        pl.kernel,
        mesh=mesh,
        out_type=jax.ShapeDtypeStruct((n_idx, EMBED), jnp.float32),
        scratch_types=[
            [pltpu.VMEM((ROWS_PER_CHUNK, IDXW), jnp.int32) for _ in range(4)],
            [pltpu.VMEM((CHUNK, EMBED), jnp.float32) for _ in range(2)],
            [pltpu.SemaphoreType.DMA for _ in range(4)],
            [pltpu.SemaphoreType.DMA for _ in range(2)],
            [pltpu.SemaphoreType.DMA for _ in range(2)],
        ],
        compiler_params=pltpu.CompilerParams(use_tc_tiling_on_sc=False),
    )
    def emb(idx_hbm, table_hbm, out_hbm, idx_v, rows_v, isem, gsem, wsem):
        wid = lax.axis_index("s") * 2 + lax.axis_index("c")
        row_base = wid * rows_per_w

        def start_idx(g, ib):
            off = row_base + g * ROWS_PER_CHUNK
            pltpu.async_copy(
                idx_hbm.at[pl.ds(off, ROWS_PER_CHUNK)], idx_v[ib], isem[ib]
            )

        def wait_idx(ib):
            pltpu.make_async_copy(
                idx_hbm.at[pl.ds(0, ROWS_PER_CHUNK)], idx_v[ib], isem[ib]
            ).wait()

        def start_gathers(ib, b):
            for j in range(ROWS_PER_CHUNK):
                pltpu.async_copy(
                    table_hbm.at[idx_v[ib].at[j]],
                    rows_v[b].at[pl.ds(j * IDXW, IDXW)],
                    gsem[b],
                )

        def wait_gathers(b):
            pltpu.make_async_copy(
                table_hbm.at[pl.ds(0, CHUNK)], rows_v[b], gsem[b]
            ).wait()

        def start_wb(g, b):
            off = (row_base + g * ROWS_PER_CHUNK) * IDXW
            pltpu.async_copy(rows_v[b], out_hbm.at[pl.ds(off, CHUNK)], wsem[b])

        def wait_wb(b):
            pltpu.make_async_copy(
                rows_v[b], out_hbm.at[pl.ds(0, CHUNK)], wsem[b]
            ).wait()

        for t in range(4):
            start_idx(jnp.int32(t), t)
        wait_idx(0)
        start_gathers(0, 0)
        wait_idx(1)
        start_gathers(1, 1)

        def loop_body(ko, carry):
            for u in range(4):
                g = ko * 4 + u
                b = u % 2
                wait_gathers(b)
                start_wb(g, b)
                start_idx(g + 4, u)
                wait_idx((u + 2) % 4)
                wait_wb(b)
                start_gathers((u + 2) % 4, b)
            return carry

        lax.fori_loop(0, (n_chunks - 4) // 4, loop_body, jnp.int32(0))

        for u in range(2):
            g = n_chunks - 4 + u
            wait_gathers(u)
            start_wb(g, u)
            wait_idx((u + 2) % 4)
            wait_wb(u)
            start_gathers((u + 2) % 4, u)
        for u in range(2):
            g = n_chunks - 2 + u
            wait_gathers(u)
            start_wb(g, u)
        for u in range(2):
            wait_wb(u)

    return emb


def kernel(x, table):
    n_idx = x.shape[0] * x.shape[1]
    idx2d = x.reshape(n_idx // IDXW, IDXW)
    out = _emb_call(n_idx)(idx2d, table)
    return out.reshape(x.shape[0], x.shape[1], EMBED)
